# initial kernel scaffold (unmeasured)
import jax
import jax.numpy as jnp
from jax import lax
from jax.experimental import pallas as pl
from jax.experimental.pallas import tpu as pltpu

N_DEV = 8
Q_CHUNK = 512


def kernel(q, k, v):
    s_per, d = q.shape
    n_chunks = s_per // Q_CHUNK
    scale = 1.0 / (d ** 0.5)

    def body(q_ref, kv_ref, o_ref, comm_ref, send_sems, recv_sems, credit_sem):
        my = lax.axis_index("i")
        left = lax.rem(my + (N_DEV - 1), N_DEV)
        right = lax.rem(my + 1, N_DEV)

        barrier = pltpu.get_barrier_semaphore()
        for nbr in (left, right):
            pl.semaphore_signal(
                barrier, inc=1,
                device_id=(nbr,), device_id_type=pl.DeviceIdType.MESH,
            )
        pl.semaphore_wait(barrier, 2)

        comm_ref[0, :, :] = kv_ref[:, :]

        m = [jnp.full((Q_CHUNK, 1), -jnp.inf, jnp.float32) for _ in range(n_chunks)]
        l = [jnp.zeros((Q_CHUNK, 1), jnp.float32) for _ in range(n_chunks)]
        acc = [jnp.zeros((Q_CHUNK, d), jnp.float32) for _ in range(n_chunks)]

        for h in range(N_DEV):
            cur = h % 2
            nxt = (h + 1) % 2
            rdma = None
            if h < N_DEV - 1:
                if h >= 1:
                    pl.semaphore_wait(credit_sem, 1)
                rdma = pltpu.make_async_remote_copy(
                    src_ref=comm_ref.at[cur],
                    dst_ref=comm_ref.at[nxt],
                    send_sem=send_sems.at[cur],
                    recv_sem=recv_sems.at[nxt],
                    device_id=(right,),
                    device_id_type=pl.DeviceIdType.MESH,
                )
                rdma.start()

            k_cur = comm_ref[cur, :s_per, :]
            v_cur = comm_ref[cur, s_per:, :]
            for c in range(n_chunks):
                qc = q_ref[pl.ds(c * Q_CHUNK, Q_CHUNK), :]
                s = lax.dot_general(
                    qc, k_cur, (((1,), (1,)), ((), ())),
                    preferred_element_type=jnp.float32,
                ) * scale
                m_new = jnp.maximum(m[c], jnp.max(s, axis=1, keepdims=True))
                alpha = jnp.exp(m[c] - m_new)
                p = jnp.exp(s - m_new)
                l[c] = l[c] * alpha + jnp.sum(p, axis=1, keepdims=True)
                acc[c] = acc[c] * alpha + lax.dot_general(
                    p.astype(jnp.bfloat16), v_cur, (((1,), (0,)), ((), ())),
                    preferred_element_type=jnp.float32,
                )
                m[c] = m_new

            if h < N_DEV - 1:
                rdma.wait()
                if h <= N_DEV - 3:
                    pl.semaphore_signal(
                        credit_sem, inc=1,
                        device_id=(left,), device_id_type=pl.DeviceIdType.MESH,
                    )

        for c in range(n_chunks):
            o_ref[pl.ds(c * Q_CHUNK, Q_CHUNK), :] = (
                acc[c] / l[c]
            ).astype(jnp.float32)

    kv = jnp.concatenate(
        [k.astype(jnp.bfloat16), v.astype(jnp.bfloat16)], axis=0
    )
    return pl.pallas_call(
        body,
        out_shape=jax.ShapeDtypeStruct((s_per, d), jnp.float32),
        in_specs=[
            pl.BlockSpec(memory_space=pltpu.VMEM),
            pl.BlockSpec(memory_space=pltpu.VMEM),
        ],
        out_specs=pl.BlockSpec(memory_space=pltpu.VMEM),
        scratch_shapes=[
            pltpu.VMEM((2, 2 * s_per, d), jnp.bfloat16),
            pltpu.SemaphoreType.DMA((2,)),
            pltpu.SemaphoreType.DMA((2,)),
            pltpu.SemaphoreType.REGULAR,
        ],
        compiler_params=pltpu.CompilerParams(collective_id=0),
    )(q.astype(jnp.bfloat16), kv)


# baseline (device time: 738354 ns/iter reference)
import jax
import jax.numpy as jnp
from jax import lax
from jax.experimental import pallas as pl
from jax.experimental.pallas import tpu as pltpu

N_DEV = 8
Q_CHUNK = 512


def kernel(q, k, v):
    s_per, d = q.shape
    n_chunks = s_per // Q_CHUNK
    scale = 1.0 / (d ** 0.5)

    def body(q_ref, kv_ref, o_ref, comm_ref, acc_ref, m_ref, l_ref,
             send_sems, recv_sems, credit_sem):
        my = lax.axis_index("i")
        left = lax.rem(my + (N_DEV - 1), N_DEV)
        right = lax.rem(my + 1, N_DEV)

        barrier = pltpu.get_barrier_semaphore()
        for nbr in (left, right):
            pl.semaphore_signal(
                barrier, inc=1,
                device_id=(nbr,), device_id_type=pl.DeviceIdType.MESH,
            )
        pl.semaphore_wait(barrier, 2)

        comm_ref[0, :, :] = kv_ref[:, :]
        m_ref[:, :] = jnp.full_like(m_ref, -jnp.inf)
        l_ref[:, :] = jnp.zeros_like(l_ref)
        acc_ref[:, :] = jnp.zeros_like(acc_ref)

        for h in range(N_DEV):
            cur = h % 2
            nxt = (h + 1) % 2
            rdma = None
            if h < N_DEV - 1:
                if h >= 1:
                    pl.semaphore_wait(credit_sem, 1)
                rdma = pltpu.make_async_remote_copy(
                    src_ref=comm_ref.at[cur],
                    dst_ref=comm_ref.at[nxt],
                    send_sem=send_sems.at[cur],
                    recv_sem=recv_sems.at[nxt],
                    device_id=(right,),
                    device_id_type=pl.DeviceIdType.MESH,
                )
                rdma.start()

            def chunk_step(c, carry, cur=cur):
                row = c * Q_CHUNK
                qc = q_ref[pl.ds(row, Q_CHUNK), :]
                k_cur = comm_ref[cur, :s_per, :]
                v_cur = comm_ref[cur, s_per:, :]
                s = lax.dot_general(
                    qc, k_cur, (((1,), (1,)), ((), ())),
                    preferred_element_type=jnp.float32,
                ) * scale
                m_prev = m_ref[pl.ds(row, Q_CHUNK), :]
                m_new = jnp.maximum(m_prev, jnp.max(s, axis=1, keepdims=True))
                alpha = jnp.exp(m_prev - m_new)
                p = jnp.exp(s - m_new)
                l_ref[pl.ds(row, Q_CHUNK), :] = (
                    l_ref[pl.ds(row, Q_CHUNK), :] * alpha
                    + jnp.sum(p, axis=1, keepdims=True)
                )
                acc_ref[pl.ds(row, Q_CHUNK), :] = (
                    acc_ref[pl.ds(row, Q_CHUNK), :] * alpha
                    + lax.dot_general(
                        p.astype(jnp.bfloat16), v_cur, (((1,), (0,)), ((), ())),
                        preferred_element_type=jnp.float32,
                    )
                )
                m_ref[pl.ds(row, Q_CHUNK), :] = m_new
                return carry

            lax.fori_loop(0, n_chunks, chunk_step, 0)

            if h < N_DEV - 1:
                rdma.wait()
                if h <= N_DEV - 3:
                    pl.semaphore_signal(
                        credit_sem, inc=1,
                        device_id=(left,), device_id_type=pl.DeviceIdType.MESH,
                    )

        o_ref[:, :] = (acc_ref[:, :] / l_ref[:, :]).astype(jnp.float32)

    kv = jnp.concatenate(
        [k.astype(jnp.bfloat16), v.astype(jnp.bfloat16)], axis=0
    )
    return pl.pallas_call(
        body,
        out_shape=jax.ShapeDtypeStruct((s_per, d), jnp.float32),
        in_specs=[
            pl.BlockSpec(memory_space=pltpu.VMEM),
            pl.BlockSpec(memory_space=pltpu.VMEM),
        ],
        out_specs=pl.BlockSpec(memory_space=pltpu.VMEM),
        scratch_shapes=[
            pltpu.VMEM((2, 2 * s_per, d), jnp.bfloat16),
            pltpu.VMEM((s_per, d), jnp.float32),
            pltpu.VMEM((s_per, 1), jnp.float32),
            pltpu.VMEM((s_per, 1), jnp.float32),
            pltpu.SemaphoreType.DMA((2,)),
            pltpu.SemaphoreType.DMA((2,)),
            pltpu.SemaphoreType.REGULAR,
        ],
        compiler_params=pltpu.CompilerParams(
            collective_id=0,
            vmem_limit_bytes=100 * 1024 * 1024,
        ),
    )(q.astype(jnp.bfloat16), kv)
